# Initial kernel scaffold; baseline (speedup 1.0000x reference)
import jax
import jax.numpy as jnp
from jax.experimental import pallas as pl


def kernel(output, mask, ind, target):
    raise NotImplementedError("write your pallas kernel here")


# speedup vs baseline: 2.5146x; 2.5146x over previous
import functools

import jax
import jax.numpy as jnp
from jax import lax
from jax.experimental import pallas as pl
from jax.experimental.pallas import tpu as pltpu
from jax.experimental.pallas import tpu_sc as plsc

_B, _C, _H, _W, _K = 32, 2, 512, 512, 500
_HW = _H * _W
_KP = 512
_NSUB = 16
_BPW = _B // _NSUB
_L = 16
_CHUNK = 128
_NCHUNK = _KP // _CHUNK

_mesh = plsc.VectorSubcoreMesh(
    core_axis_name="c", subcore_axis_name="s", num_cores=1
)


@functools.partial(
    pl.kernel,
    out_type=jax.ShapeDtypeStruct((_L,), jnp.float32),
    mesh=_mesh,
    scratch_types=[
        pltpu.VMEM((_KP,), jnp.int32),
        pltpu.VMEM((2, _KP), jnp.int32),
        pltpu.VMEM((2, _KP), jnp.float32),
        pltpu.VMEM((_KP,), jnp.float32),
        pltpu.VMEM((2, _KP), jnp.float32),
        pltpu.VMEM((2, _L), jnp.float32),
        pltpu.VMEM((2 * _NSUB, _L), jnp.float32),
        pltpu.VMEM((2 * _L,), jnp.float32),
        pltpu.VMEM((2 * _L,), jnp.float32),
        pltpu.VMEM((_L,), jnp.float32),
        pltpu.VMEM_SHARED((2 * _NSUB, _L), jnp.float32),
        pltpu.SemaphoreType.DMA,
    ],
)
def _sc_loss(flat_hbm, ind_hbm, mask_hbm, tgt_hbm, out_hbm,
             ind_v, idx_v, pred_v, mask_v, tgt_v, part_v, all_v, red_l, red_m,
             res_v, parts_sh, sem):
    sid = lax.axis_index("s")

    loss_acc = jnp.zeros((_L,), jnp.float32)
    m_acc = jnp.zeros((_L,), jnp.float32)

    for j in range(_BPW):
        b = sid * _BPW + j
        base0 = b * (_C * _HW)

        pltpu.sync_copy(ind_hbm.at[b], ind_v)
        pltpu.sync_copy(mask_hbm.at[b], mask_v)
        pltpu.sync_copy(tgt_hbm.at[b], tgt_v)

        for i in range(_KP // _L):
            iv = ind_v[pl.ds(i * _L, _L)]
            idx_v[0, pl.ds(i * _L, _L)] = iv + base0
            idx_v[1, pl.ds(i * _L, _L)] = iv + (base0 + _HW)

        copies = []
        for c in range(_C):
            for q in range(_NCHUNK):
                sl = pl.ds(q * _CHUNK, _CHUNK)
                copies.append(
                    pltpu.async_copy(
                        flat_hbm.at[idx_v.at[c, sl]], pred_v.at[c, sl], sem
                    )
                )
        for cp in copies:
            cp.wait()

        for i in range(_KP // _L):
            sl = pl.ds(i * _L, _L)
            m = mask_v[sl]
            d0 = (pred_v[0, sl] - tgt_v[0, sl]) * m
            d1 = (pred_v[1, sl] - tgt_v[1, sl]) * m
            loss_acc = loss_acc + (jnp.abs(d0) + jnp.abs(d1))
            m_acc = m_acc + m

    part_v[0, :] = loss_acc
    part_v[1, :] = m_acc
    pltpu.sync_copy(part_v, parts_sh.at[pl.ds(sid * 2, 2)])
    plsc.subcore_barrier()

    pltpu.sync_copy(parts_sh, all_v)
    lv = jnp.zeros((_L,), jnp.float32)
    mv = jnp.zeros((_L,), jnp.float32)
    for t in range(_NSUB):
        lv = lv + all_v[2 * t, :]
        mv = mv + all_v[2 * t + 1, :]
    zero = jnp.zeros((_L,), jnp.float32)
    red_l[pl.ds(_L, _L)] = zero
    red_m[pl.ds(_L, _L)] = zero
    for sh in (8, 4, 2, 1):
        red_l[pl.ds(0, _L)] = lv
        red_m[pl.ds(0, _L)] = mv
        lv = lv + red_l[pl.ds(sh, _L)]
        mv = mv + red_m[pl.ds(sh, _L)]
    res_v[...] = lv / (mv * float(_C) + 0.0001)

    @pl.when(sid == 0)
    def _():
        pltpu.sync_copy(res_v, out_hbm)


def kernel(output, mask, ind, target):
    flat = output.reshape(-1)
    pad = _KP - _K
    ind_p = jnp.pad(ind.reshape(_B, _K).astype(jnp.int32), ((0, 0), (0, pad)))
    mask_p = jnp.pad(mask.reshape(_B, _K), ((0, 0), (0, pad)))
    tgt_p = jnp.pad(jnp.transpose(target, (0, 2, 1)), ((0, 0), (0, 0), (0, pad)))
    out = _sc_loss(flat, ind_p, mask_p, tgt_p)
    return out[0]


# speedup vs baseline: 7.0578x; 2.8067x over previous
import functools

import jax
import jax.numpy as jnp
from jax import lax
from jax.experimental import pallas as pl
from jax.experimental.pallas import tpu as pltpu
from jax.experimental.pallas import tpu_sc as plsc

_B, _C, _H, _W, _K = 32, 2, 512, 512, 500
_HW = _H * _W
_KP = 512
_NSUB = 16
_BPW = _B // _NSUB
_L = 16
_CHUNK = 128
_NCHUNK = _KP // _CHUNK

_mesh = plsc.VectorSubcoreMesh(
    core_axis_name="c", subcore_axis_name="s", num_cores=1
)


@functools.partial(
    pl.kernel,
    out_type=jax.ShapeDtypeStruct((_L,), jnp.float32),
    mesh=_mesh,
    scratch_types=[
        pltpu.VMEM((_KP,), jnp.int32),
        pltpu.VMEM((2, _KP), jnp.int32),
        pltpu.VMEM((2, _KP), jnp.float32),
        pltpu.VMEM((_KP,), jnp.float32),
        pltpu.VMEM((2, _KP), jnp.float32),
        pltpu.VMEM((2, _L), jnp.float32),
        pltpu.VMEM((2 * _NSUB, _L), jnp.float32),
        pltpu.VMEM((2 * _L,), jnp.float32),
        pltpu.VMEM((2 * _L,), jnp.float32),
        pltpu.VMEM((_L,), jnp.float32),
        pltpu.VMEM_SHARED((2 * _NSUB, _L), jnp.float32),
        pltpu.SemaphoreType.DMA,
    ],
)
def _sc_loss(flat_hbm, ind_hbm, mask_hbm, tgt_hbm, out_hbm,
             ind_v, idx_v, pred_v, mask_v, tgt_v, part_v, all_v, red_l, red_m,
             res_v, parts_sh, sem):
    sid = lax.axis_index("s")

    loss_acc = jnp.zeros((_L,), jnp.float32)
    m_acc = jnp.zeros((_L,), jnp.float32)

    for j in range(_BPW):
        b = sid * _BPW + j
        base0 = b * (_C * _HW)

        pltpu.sync_copy(ind_hbm.at[b], ind_v)
        pltpu.sync_copy(mask_hbm.at[b], mask_v)
        pltpu.sync_copy(tgt_hbm.at[b], tgt_v)

        for i in range(_KP // _L):
            iv = ind_v[pl.ds(i * _L, _L)]
            h = jax.lax.shift_right_logical(iv, 9)
            w = jnp.bitwise_and(iv, _W - 1)
            off = (
                jax.lax.shift_left(jax.lax.shift_right_logical(h, 3), 12)
                + jax.lax.shift_left(jax.lax.shift_right_logical(w, 7), 10)
                + jax.lax.shift_left(jnp.bitwise_and(h, 7), 7)
                + jnp.bitwise_and(w, 127)
            )
            idx_v[0, pl.ds(i * _L, _L)] = off + base0
            idx_v[1, pl.ds(i * _L, _L)] = off + (base0 + _HW)

        copies = []
        for c in range(_C):
            for q in range(_NCHUNK):
                sl = pl.ds(q * _CHUNK, _CHUNK)
                copies.append(
                    pltpu.async_copy(
                        flat_hbm.at[idx_v.at[c, sl]], pred_v.at[c, sl], sem
                    )
                )
        for cp in copies:
            cp.wait()

        for i in range(_KP // _L):
            sl = pl.ds(i * _L, _L)
            m = mask_v[sl]
            d0 = (pred_v[0, sl] - tgt_v[0, sl]) * m
            d1 = (pred_v[1, sl] - tgt_v[1, sl]) * m
            loss_acc = loss_acc + (jnp.abs(d0) + jnp.abs(d1))
            m_acc = m_acc + m

    part_v[0, :] = loss_acc
    part_v[1, :] = m_acc
    pltpu.sync_copy(part_v, parts_sh.at[pl.ds(sid * 2, 2)])
    plsc.subcore_barrier()

    pltpu.sync_copy(parts_sh, all_v)
    lv = jnp.zeros((_L,), jnp.float32)
    mv = jnp.zeros((_L,), jnp.float32)
    for t in range(_NSUB):
        lv = lv + all_v[2 * t, :]
        mv = mv + all_v[2 * t + 1, :]
    zero = jnp.zeros((_L,), jnp.float32)
    red_l[pl.ds(_L, _L)] = zero
    red_m[pl.ds(_L, _L)] = zero
    for sh in (8, 4, 2, 1):
        red_l[pl.ds(0, _L)] = lv
        red_m[pl.ds(0, _L)] = mv
        lv = lv + red_l[pl.ds(sh, _L)]
        mv = mv + red_m[pl.ds(sh, _L)]
    res_v[...] = lv / (mv * float(_C) + 0.0001)

    @pl.when(sid == 0)
    def _():
        pltpu.sync_copy(res_v, out_hbm)


def kernel(output, mask, ind, target):
    t6 = output.reshape(_B, _C, _H // 8, 8, _W // 128, 128)
    t6 = jnp.transpose(t6, (0, 1, 2, 4, 3, 5))
    flat = t6.reshape(-1)
    pad = _KP - _K
    ind_p = jnp.pad(ind.reshape(_B, _K).astype(jnp.int32), ((0, 0), (0, pad)))
    mask_p = jnp.pad(mask.reshape(_B, _K), ((0, 0), (0, pad)))
    tgt_p = jnp.pad(jnp.transpose(target, (0, 2, 1)), ((0, 0), (0, 0), (0, pad)))
    out = _sc_loss(flat, ind_p, mask_p, tgt_p)
    return out[0]


# speedup vs baseline: 8.2549x; 1.1696x over previous
import functools

import jax
import jax.numpy as jnp
from jax import lax
from jax.experimental import pallas as pl
from jax.experimental.pallas import tpu as pltpu
from jax.experimental.pallas import tpu_sc as plsc

_B, _C, _H, _W, _K = 32, 2, 512, 512, 500
_HW = _H * _W
_KP = 512
_NSUB = 16
_BPW = _B // _NSUB
_L = 16
_CHUNK = 128
_NCHUNK = _KP // _CHUNK

_mesh = plsc.VectorSubcoreMesh(
    core_axis_name="c", subcore_axis_name="s", num_cores=1
)


@functools.partial(
    pl.kernel,
    out_type=jax.ShapeDtypeStruct((_L,), jnp.float32),
    mesh=_mesh,
    scratch_types=[
        pltpu.VMEM((_BPW, _KP), jnp.int32),
        pltpu.VMEM((_BPW, 2, _KP), jnp.int32),
        pltpu.VMEM((_BPW, 2, _KP), jnp.float32),
        pltpu.VMEM((_BPW, _KP), jnp.float32),
        pltpu.VMEM((_BPW, 2, _KP), jnp.float32),
        pltpu.VMEM((2, _L), jnp.float32),
        pltpu.VMEM((2 * _NSUB, _L), jnp.float32),
        pltpu.VMEM((2 * _L,), jnp.float32),
        pltpu.VMEM((2 * _L,), jnp.float32),
        pltpu.VMEM((_L,), jnp.float32),
        pltpu.VMEM_SHARED((2 * _NSUB, _L), jnp.float32),
        pltpu.SemaphoreType.DMA,
        pltpu.SemaphoreType.DMA,
        pltpu.SemaphoreType.DMA,
        pltpu.SemaphoreType.DMA,
        pltpu.SemaphoreType.DMA,
        pltpu.SemaphoreType.DMA,
    ],
)
def _sc_loss(flat_hbm, ind_hbm, mask_hbm, tgt_hbm, out_hbm,
             ind_v, idx_v, pred_v, mask_v, tgt_v, part_v, all_v, red_l, red_m,
             res_v, parts_sh, sem_i0, sem_i1, sem_mt0, sem_mt1, sem_g0, sem_g1):
    sid = lax.axis_index("s")
    b0 = sid * _BPW

    sem_i = (sem_i0, sem_i1)
    sem_mt = (sem_mt0, sem_mt1)
    sem_g = (sem_g0, sem_g1)
    ind_cp = [pltpu.async_copy(ind_hbm.at[b0 + j], ind_v.at[j], sem_i[j])
              for j in range(_BPW)]
    mt_cp = [(pltpu.async_copy(mask_hbm.at[b0 + j], mask_v.at[j], sem_mt[j]),
              pltpu.async_copy(tgt_hbm.at[b0 + j], tgt_v.at[j], sem_mt[j]))
             for j in range(_BPW)]

    gathers = [[], []]
    for j in range(_BPW):
        ind_cp[j].wait()
        base0 = (b0 + j) * (_C * _HW)
        for i in range(_KP // _L):
            iv = ind_v[j, pl.ds(i * _L, _L)]
            h = jax.lax.shift_right_logical(iv, 9)
            w = jnp.bitwise_and(iv, _W - 1)
            off = (
                jax.lax.shift_left(jax.lax.shift_right_logical(h, 3), 12)
                + jax.lax.shift_left(jax.lax.shift_right_logical(w, 7), 10)
                + jax.lax.shift_left(jnp.bitwise_and(h, 7), 7)
                + jnp.bitwise_and(w, 127)
            )
            idx_v[j, 0, pl.ds(i * _L, _L)] = off + base0
            idx_v[j, 1, pl.ds(i * _L, _L)] = off + (base0 + _HW)
        for c in range(_C):
            for q in range(_NCHUNK):
                sl = pl.ds(q * _CHUNK, _CHUNK)
                gathers[j].append(
                    pltpu.async_copy(
                        flat_hbm.at[idx_v.at[j, c, sl]],
                        pred_v.at[j, c, sl], sem_g[j]
                    )
                )

    loss_acc = jnp.zeros((_L,), jnp.float32)
    m_acc = jnp.zeros((_L,), jnp.float32)
    for j in range(_BPW):
        for cp in gathers[j]:
            cp.wait()
        mt_cp[j][0].wait()
        mt_cp[j][1].wait()
        for i in range(_KP // _L):
            sl = pl.ds(i * _L, _L)
            m = mask_v[j, sl]
            d0 = (pred_v[j, 0, sl] - tgt_v[j, 0, sl]) * m
            d1 = (pred_v[j, 1, sl] - tgt_v[j, 1, sl]) * m
            loss_acc = loss_acc + (jnp.abs(d0) + jnp.abs(d1))
            m_acc = m_acc + m

    part_v[0, :] = loss_acc
    part_v[1, :] = m_acc
    pltpu.sync_copy(part_v, parts_sh.at[pl.ds(sid * 2, 2)])
    plsc.subcore_barrier()

    pltpu.sync_copy(parts_sh, all_v)
    lv = jnp.zeros((_L,), jnp.float32)
    mv = jnp.zeros((_L,), jnp.float32)
    for t in range(_NSUB):
        lv = lv + all_v[2 * t, :]
        mv = mv + all_v[2 * t + 1, :]
    zero = jnp.zeros((_L,), jnp.float32)
    red_l[pl.ds(_L, _L)] = zero
    red_m[pl.ds(_L, _L)] = zero
    for sh in (8, 4, 2, 1):
        red_l[pl.ds(0, _L)] = lv
        red_m[pl.ds(0, _L)] = mv
        lv = lv + red_l[pl.ds(sh, _L)]
        mv = mv + red_m[pl.ds(sh, _L)]
    res_v[...] = lv / (mv * float(_C) + 0.0001)

    @pl.when(sid == 0)
    def _():
        pltpu.sync_copy(res_v, out_hbm)


def kernel(output, mask, ind, target):
    t6 = output.reshape(_B, _C, _H // 8, 8, _W // 128, 128)
    t6 = jnp.transpose(t6, (0, 1, 2, 4, 3, 5))
    flat = t6.reshape(-1)
    pad = _KP - _K
    ind_p = jnp.pad(ind.reshape(_B, _K).astype(jnp.int32), ((0, 0), (0, pad)))
    mask_p = jnp.pad(mask.reshape(_B, _K), ((0, 0), (0, pad)))
    tgt_p = jnp.pad(jnp.transpose(target, (0, 2, 1)), ((0, 0), (0, 0), (0, pad)))
    out = _sc_loss(flat, ind_p, mask_p, tgt_p)
    return out[0]


# speedup vs baseline: 8.3581x; 1.0125x over previous
import functools

import jax
import jax.numpy as jnp
from jax import lax
from jax.experimental import pallas as pl
from jax.experimental.pallas import tpu as pltpu
from jax.experimental.pallas import tpu_sc as plsc

_B, _C, _H, _W, _K = 32, 2, 512, 512, 500
_HW = _H * _W
_KP = 512
_NSUB = 16
_BPW = _B // _NSUB
_L = 16
_CHUNK = 128
_NCHUNK = _KP // _CHUNK
_ROW = 4 * _KP

_mesh = plsc.VectorSubcoreMesh(
    core_axis_name="c", subcore_axis_name="s", num_cores=1
)


@functools.partial(
    pl.kernel,
    out_type=jax.ShapeDtypeStruct((_L,), jnp.float32),
    mesh=_mesh,
    scratch_types=[
        pltpu.VMEM((_ROW,), jnp.float32),
        pltpu.VMEM((_ROW,), jnp.float32),
        pltpu.VMEM((_KP,), jnp.int32),
        pltpu.VMEM((_KP,), jnp.int32),
        pltpu.VMEM((_BPW, 2, _KP), jnp.int32),
        pltpu.VMEM((_BPW, 2, _KP), jnp.float32),
        pltpu.VMEM((2, _L), jnp.float32),
        pltpu.VMEM((2 * _NSUB, _L), jnp.float32),
        pltpu.VMEM((2 * _L,), jnp.float32),
        pltpu.VMEM((2 * _L,), jnp.float32),
        pltpu.VMEM((_L,), jnp.float32),
        pltpu.VMEM_SHARED((2 * _NSUB, _L), jnp.float32),
        pltpu.SemaphoreType.DMA,
        pltpu.SemaphoreType.DMA,
        pltpu.SemaphoreType.DMA,
        pltpu.SemaphoreType.DMA,
    ],
)
def _sc_loss(flat_hbm, packed_hbm, packedi_hbm, out_hbm,
             row0_v, row1_v, ind0_v, ind1_v, idx_v, pred_v, part_v, all_v,
             red_l, red_m, res_v, parts_sh, sem_r0, sem_r1, sem_g0, sem_g1):
    sid = lax.axis_index("s")
    b0 = sid * _BPW
    rows = (row0_v, row1_v)
    inds = (ind0_v, ind1_v)
    sem_r = (sem_r0, sem_r1)
    sem_g = (sem_g0, sem_g1)

    row_cp = [(pltpu.async_copy(packed_hbm.at[b0 + j], rows[j], sem_r[j]),
               pltpu.async_copy(packedi_hbm.at[b0 + j, pl.ds(_KP, _KP)],
                                inds[j], sem_r[j]))
              for j in range(_BPW)]

    gathers = [[], []]
    for j in range(_BPW):
        row_cp[j][0].wait()
        row_cp[j][1].wait()
        base0 = (b0 + j) * (_C * _HW)
        for i in range(_KP // _L):
            iv = inds[j][pl.ds(i * _L, _L)]
            h = jax.lax.shift_right_logical(iv, 9)
            w = jnp.bitwise_and(iv, _W - 1)
            off = (
                jax.lax.shift_left(jax.lax.shift_right_logical(h, 3), 12)
                + jax.lax.shift_left(jax.lax.shift_right_logical(w, 7), 10)
                + jax.lax.shift_left(jnp.bitwise_and(h, 7), 7)
                + jnp.bitwise_and(w, 127)
            )
            idx_v[j, 0, pl.ds(i * _L, _L)] = off + base0
            idx_v[j, 1, pl.ds(i * _L, _L)] = off + (base0 + _HW)
        for c in range(_C):
            for q in range(_NCHUNK):
                sl = pl.ds(q * _CHUNK, _CHUNK)
                gathers[j].append(
                    pltpu.async_copy(
                        flat_hbm.at[idx_v.at[j, c, sl]],
                        pred_v.at[j, c, sl], sem_g[j]
                    )
                )

    loss_acc = jnp.zeros((_L,), jnp.float32)
    m_acc = jnp.zeros((_L,), jnp.float32)
    for j in range(_BPW):
        for cp in gathers[j]:
            cp.wait()
        for i in range(_KP // _L):
            sl = pl.ds(i * _L, _L)
            m = rows[j][pl.ds(i * _L, _L)]
            t0 = rows[j][pl.ds(2 * _KP + i * _L, _L)]
            t1 = rows[j][pl.ds(3 * _KP + i * _L, _L)]
            d0 = (pred_v[j, 0, sl] - t0) * m
            d1 = (pred_v[j, 1, sl] - t1) * m
            loss_acc = loss_acc + (jnp.abs(d0) + jnp.abs(d1))
            m_acc = m_acc + m

    part_v[0, :] = loss_acc
    part_v[1, :] = m_acc
    pltpu.sync_copy(part_v, parts_sh.at[pl.ds(sid * 2, 2)])
    plsc.subcore_barrier()

    pltpu.sync_copy(parts_sh, all_v)
    lv = jnp.zeros((_L,), jnp.float32)
    mv = jnp.zeros((_L,), jnp.float32)
    for t in range(_NSUB):
        lv = lv + all_v[2 * t, :]
        mv = mv + all_v[2 * t + 1, :]
    zero = jnp.zeros((_L,), jnp.float32)
    red_l[pl.ds(_L, _L)] = zero
    red_m[pl.ds(_L, _L)] = zero
    for sh in (8, 4, 2, 1):
        red_l[pl.ds(0, _L)] = lv
        red_m[pl.ds(0, _L)] = mv
        lv = lv + red_l[pl.ds(sh, _L)]
        mv = mv + red_m[pl.ds(sh, _L)]
    res_v[...] = lv / (mv * float(_C) + 0.0001)

    @pl.when(sid == 0)
    def _():
        pltpu.sync_copy(res_v, out_hbm)


def kernel(output, mask, ind, target):
    t6 = output.reshape(_B, _C, _H // 8, 8, _W // 128, 128)
    t6 = jnp.transpose(t6, (0, 1, 2, 4, 3, 5))
    flat = t6.reshape(-1)
    pad = _KP - _K
    maskp = jnp.pad(mask.reshape(_B, _K), ((0, 0), (0, pad)))
    indf = lax.bitcast_convert_type(
        jnp.pad(ind.reshape(_B, _K), ((0, 0), (0, pad))), jnp.float32)
    tgtp = jnp.pad(jnp.transpose(target, (0, 2, 1)),
                   ((0, 0), (0, 0), (0, pad))).reshape(_B, 2 * _KP)
    packed = jnp.concatenate([maskp, indf, tgtp], axis=1)
    packed_i = lax.bitcast_convert_type(packed, jnp.int32)
    out = _sc_loss(flat, packed, packed_i)
    return out[0]


# speedup vs baseline: 8.4940x; 1.0163x over previous
import functools

import jax
import jax.numpy as jnp
from jax import lax
from jax.experimental import pallas as pl
from jax.experimental.pallas import tpu as pltpu
from jax.experimental.pallas import tpu_sc as plsc

_B, _C, _H, _W, _K = 32, 2, 512, 512, 500
_HW = _H * _W
_KP = 512
_NSUB = 16
_BPW = _B // _NSUB
_L = 16
_CHUNK = 128
_NCHUNK = _KP // _CHUNK
_ROW = 4 * _KP

_mesh = plsc.VectorSubcoreMesh(
    core_axis_name="c", subcore_axis_name="s", num_cores=1
)


@functools.partial(
    pl.kernel,
    out_type=jax.ShapeDtypeStruct((_L,), jnp.float32),
    mesh=_mesh,
    scratch_types=[
        pltpu.VMEM((_ROW,), jnp.float32),
        pltpu.VMEM((_ROW,), jnp.float32),
        pltpu.VMEM((_KP,), jnp.int32),
        pltpu.VMEM((_KP,), jnp.int32),
        pltpu.VMEM((_BPW, 2, _KP), jnp.int32),
        pltpu.VMEM((_BPW, 2, _KP), jnp.float32),
        pltpu.VMEM((2, _L), jnp.float32),
        pltpu.VMEM((2 * _NSUB, _L), jnp.float32),
        pltpu.VMEM((2 * _L,), jnp.float32),
        pltpu.VMEM((2 * _L,), jnp.float32),
        pltpu.VMEM((_L,), jnp.float32),
        pltpu.VMEM_SHARED((2 * _NSUB, _L), jnp.float32),
        pltpu.SemaphoreType.DMA,
        pltpu.SemaphoreType.DMA,
        pltpu.SemaphoreType.DMA,
        pltpu.SemaphoreType.DMA,
    ],
)
def _sc_loss(flat_hbm, packed_hbm, packedi_hbm, out_hbm,
             row0_v, row1_v, ind0_v, ind1_v, idx_v, pred_v, part_v, all_v,
             red_l, red_m, res_v, parts_sh, sem_r0, sem_r1, sem_g0, sem_g1):
    sid = lax.axis_index("s")
    b0 = sid * _BPW
    rows = (row0_v, row1_v)
    inds = (ind0_v, ind1_v)
    sem_r = (sem_r0, sem_r1)
    sem_g = (sem_g0, sem_g1)

    row_cp = [(pltpu.async_copy(packed_hbm.at[b0 + j], rows[j], sem_r[j]),
               pltpu.async_copy(packedi_hbm.at[b0 + j, pl.ds(_KP, _KP)],
                                inds[j], sem_r[j]))
              for j in range(_BPW)]

    gathers = [[], []]
    for j in range(_BPW):
        row_cp[j][0].wait()
        row_cp[j][1].wait()
        base0 = (b0 + j) * (_C * _HW)
        ind_ref = inds[j]

        def idx_body(i, _, j=j, ind_ref=ind_ref, base0=base0):
            o = i * _L
            iv = ind_ref[pl.ds(o, _L)]
            h = jax.lax.shift_right_logical(iv, 9)
            w = jnp.bitwise_and(iv, _W - 1)
            off = (
                jax.lax.shift_left(jax.lax.shift_right_logical(h, 3), 12)
                + jax.lax.shift_left(jax.lax.shift_right_logical(w, 7), 10)
                + jax.lax.shift_left(jnp.bitwise_and(h, 7), 7)
                + jnp.bitwise_and(w, 127)
            )
            idx_v[j, 0, pl.ds(o, _L)] = off + base0
            idx_v[j, 1, pl.ds(o, _L)] = off + (base0 + _HW)
            return 0

        lax.fori_loop(0, _KP // _L, idx_body, 0, unroll=2)
        for c in range(_C):
            for q in range(_NCHUNK):
                sl = pl.ds(q * _CHUNK, _CHUNK)
                gathers[j].append(
                    pltpu.async_copy(
                        flat_hbm.at[idx_v.at[j, c, sl]],
                        pred_v.at[j, c, sl], sem_g[j]
                    )
                )

    loss_acc = jnp.zeros((_L,), jnp.float32)
    m_acc = jnp.zeros((_L,), jnp.float32)
    for j in range(_BPW):
        for cp in gathers[j]:
            cp.wait()
        row_ref = rows[j]

        def acc_body(i, carry, j=j, row_ref=row_ref):
            la, ma = carry
            o = i * _L
            sl = pl.ds(o, _L)
            m = row_ref[pl.ds(o, _L)]
            t0 = row_ref[pl.ds(2 * _KP + o, _L)]
            t1 = row_ref[pl.ds(3 * _KP + o, _L)]
            d0 = (pred_v[j, 0, sl] - t0) * m
            d1 = (pred_v[j, 1, sl] - t1) * m
            return la + (jnp.abs(d0) + jnp.abs(d1)), ma + m

        loss_acc, m_acc = lax.fori_loop(
            0, _KP // _L, acc_body, (loss_acc, m_acc), unroll=2)

    part_v[0, :] = loss_acc
    part_v[1, :] = m_acc
    pltpu.sync_copy(part_v, parts_sh.at[pl.ds(sid * 2, 2)])
    plsc.subcore_barrier()

    pltpu.sync_copy(parts_sh, all_v)
    lv = jnp.zeros((_L,), jnp.float32)
    mv = jnp.zeros((_L,), jnp.float32)
    for t in range(_NSUB):
        lv = lv + all_v[2 * t, :]
        mv = mv + all_v[2 * t + 1, :]
    zero = jnp.zeros((_L,), jnp.float32)
    red_l[pl.ds(_L, _L)] = zero
    red_m[pl.ds(_L, _L)] = zero
    for sh in (8, 4, 2, 1):
        red_l[pl.ds(0, _L)] = lv
        red_m[pl.ds(0, _L)] = mv
        lv = lv + red_l[pl.ds(sh, _L)]
        mv = mv + red_m[pl.ds(sh, _L)]
    res_v[...] = lv / (mv * float(_C) + 0.0001)

    @pl.when(sid == 0)
    def _():
        pltpu.sync_copy(res_v, out_hbm)


def kernel(output, mask, ind, target):
    t6 = output.reshape(_B, _C, _H // 8, 8, _W // 128, 128)
    t6 = jnp.transpose(t6, (0, 1, 2, 4, 3, 5))
    flat = t6.reshape(-1)
    pad = _KP - _K
    maskp = jnp.pad(mask.reshape(_B, _K), ((0, 0), (0, pad)))
    indf = lax.bitcast_convert_type(
        jnp.pad(ind.reshape(_B, _K), ((0, 0), (0, pad))), jnp.float32)
    tgtp = jnp.pad(jnp.transpose(target, (0, 2, 1)),
                   ((0, 0), (0, 0), (0, pad))).reshape(_B, 2 * _KP)
    packed = jnp.concatenate([maskp, indf, tgtp], axis=1)
    packed_i = lax.bitcast_convert_type(packed, jnp.int32)
    out = _sc_loss(flat, packed, packed_i)
    return out[0]
